# Initial kernel scaffold; baseline (speedup 1.0000x reference)
import jax
import jax.numpy as jnp
from jax.experimental import pallas as pl


def kernel(x, edge_index, edge_attr, graph_attr, W1, b1, Wg, bg, Wl1, bl1, Wl2, bl2):
    raise NotImplementedError("write your pallas kernel here")


# speedup vs baseline: 14.0963x; 14.0963x over previous
import jax
import jax.numpy as jnp
from jax import lax
from jax.experimental import pallas as pl
from jax.experimental.pallas import tpu as pltpu
from jax.experimental.pallas import tpu_sc as plsc

N = 10000
E = 160000
F_IN = 256
FH = 128
H = 512

NSC = 2
NT = 16
CKD = 1000
CKA = 200
EPT = E // NT
SL = 640


def _sc_body(x2_hbm, src_hbm, dst_hbm, w_hbm, s2_hbm, dinv_hbm,
             S_sp, deg_sp, dinv_sp, dinv_v, degsl, dstd, wd,
             srca, dsta, wa, cv, rows, sem):
    c = lax.axis_index("c")
    s = lax.axis_index("s")

    def _zdeg(i, _):
        degsl[pl.ds(i * 16, 16)] = jnp.zeros((16,), jnp.float32)
        return ()
    lax.fori_loop(0, SL // 16, _zdeg, ())
    soff = jnp.minimum(s * SL, N - SL)
    pltpu.sync_copy(degsl, deg_sp.at[pl.ds(soff, SL)])

    def _zrow(r, _):
        for f in range(8):
            rows[r, pl.ds(f * 16, 16)] = jnp.zeros((16,), jnp.float32)
        return ()
    lax.fori_loop(0, CKA, _zrow, ())
    for o in (0, 160, 320, 480):
        pltpu.sync_copy(rows.at[pl.ds(0, 160)],
                        S_sp.at[pl.ds(soff + o, 160)])
    plsc.subcore_barrier()

    dbase = s * EPT
    def _dchunk(g, _):
        off = dbase + g * CKD
        pltpu.sync_copy(dst_hbm.at[pl.ds(off, CKD)], dstd)
        pltpu.sync_copy(w_hbm.at[pl.ds(off, CKD)], wd)
        pltpu.sync_copy(wd, deg_sp.at[dstd], add=True)
        return ()
    lax.fori_loop(0, EPT // CKD, _dchunk, ())
    plsc.subcore_barrier()

    pltpu.sync_copy(deg_sp.at[pl.ds(soff, SL)], degsl)
    def _newton(i, _):
        d = degsl[pl.ds(i * 16, 16)] + 1.0
        half = 0.5 * d
        ib = lax.bitcast_convert_type(d, jnp.int32)
        ib = jnp.int32(0x5F3759DF) - lax.shift_right_logical(ib, 1)
        r = lax.bitcast_convert_type(ib, jnp.float32)
        for _ in range(4):
            r = r * (1.5 - half * r * r)
        degsl[pl.ds(i * 16, 16)] = r
        return ()
    lax.fori_loop(0, SL // 16, _newton, ())
    pltpu.sync_copy(degsl, dinv_sp.at[pl.ds(soff, SL)])
    @pl.when(c == 0)
    def _():
        pltpu.sync_copy(degsl, dinv_hbm.at[pl.ds(soff, SL)])
    plsc.subcore_barrier()
    pltpu.sync_copy(dinv_sp, dinv_v)

    def _agg_chunk(off, srcb, dstb, wb, cvb, rowsb, ckn):
        pltpu.sync_copy(src_hbm.at[pl.ds(off, ckn)], srcb)
        pltpu.sync_copy(dst_hbm.at[pl.ds(off, ckn)], dstb)
        pltpu.sync_copy(w_hbm.at[pl.ds(off, ckn)], wb)
        gat = pltpu.async_copy(x2_hbm.at[c].at[srcb], rowsb, sem)
        def _cb(i, _):
            sv = srcb[pl.ds(i * 16, 16)]
            dvec = plsc.load_gather(dinv_v, [sv])
            cvb[pl.ds(i * 16, 16)] = wb[pl.ds(i * 16, 16)] * dvec
            return ()
        lax.fori_loop(0, ckn // 16, _cb, ())
        if ckn % 16:
            o = ckn - 16
            sv = srcb[pl.ds(o, 16)]
            dvec = plsc.load_gather(dinv_v, [sv])
            cvb[pl.ds(o, 16)] = wb[pl.ds(o, 16)] * dvec
        gat.wait()
        def _scale(e, _):
            cs = plsc.load_gather(cvb, [jnp.full((16,), e, jnp.int32)])
            for f in range(8):
                sl = (e, pl.ds(f * 16, 16))
                rows[sl] = rows[sl] * cs
            return ()
        lax.fori_loop(0, ckn, _scale, ())
        pltpu.sync_copy(rowsb, S_sp.at[dstb], add=True)

    def _chunk(g, _):
        _agg_chunk(dbase + g * CKA, srca, dsta, wa, cv, rows, CKA)
        return ()
    lax.fori_loop(0, EPT // CKA, _chunk, ())
    plsc.subcore_barrier()

    pltpu.sync_copy(S_sp.at[pl.ds(soff, SL)],
                    s2_hbm.at[c].at[pl.ds(soff, SL)])


def _sc_aggregate(x2, src, dst, w):
    mesh = plsc.VectorSubcoreMesh(core_axis_name="c", subcore_axis_name="s")
    return pl.kernel(
        _sc_body,
        out_type=[
            jax.ShapeDtypeStruct((NSC, N, FH), jnp.float32),
            jax.ShapeDtypeStruct((N,), jnp.float32),
        ],
        mesh=mesh,
        compiler_params=pltpu.CompilerParams(needs_layout_passes=False),
        scratch_types=[
            pltpu.VMEM_SHARED((N, FH), jnp.float32),
            pltpu.VMEM_SHARED((N,), jnp.float32),
            pltpu.VMEM_SHARED((N,), jnp.float32),
            pltpu.VMEM((N,), jnp.float32),
            pltpu.VMEM((SL,), jnp.float32),
            pltpu.VMEM((CKD,), jnp.int32),
            pltpu.VMEM((CKD,), jnp.float32),
            pltpu.VMEM((CKA,), jnp.int32),
            pltpu.VMEM((CKA,), jnp.int32),
            pltpu.VMEM((CKA,), jnp.float32),
            pltpu.VMEM((CKA,), jnp.float32),
            pltpu.VMEM((CKA, FH), jnp.float32),
            pltpu.SemaphoreType.DMA,
        ],
    )(x2, src, dst, w)


BND = 2000


def _tc_body(s2, x, dinv, W1r, b1r, gar, Wgr, bgr, Wl1r, bl1r, Wl2r, bl2r,
             out, acc):
    i = pl.program_id(0)

    @pl.when(i == 0)
    def _():
        acc[...] = jnp.zeros_like(acc)

    dv = dinv[...]
    t = jnp.concatenate([s2[0], s2[1]], axis=1)
    agg = dv * (t + dv * x[...])
    h = jnp.dot(agg, W1r[...], preferred_element_type=jnp.float32) + b1r[...]
    h = jnp.maximum(h, 0.0)
    acc[...] += jnp.sum(h, axis=0, keepdims=True)

    @pl.when(i == pl.num_programs(0) - 1)
    def _():
        hm = acc[...] / N
        g = jnp.dot(gar[...], Wgr[...], preferred_element_type=jnp.float32)
        g = jnp.maximum(g + bgr[...], 0.0)
        z = jnp.concatenate([hm, g], axis=1)
        z1 = jnp.dot(z, Wl1r[...], preferred_element_type=jnp.float32)
        z1 = jnp.maximum(z1 + bl1r[...], 0.0)
        z2 = jnp.dot(z1, Wl2r[...], preferred_element_type=jnp.float32)
        z2 = z2 + bl2r[...]
        m = jnp.max(z2, axis=1, keepdims=True)
        lse = m + jnp.log(jnp.sum(jnp.exp(z2 - m), axis=1, keepdims=True))
        out[...] = z2 - lse


def _tc_head(s2, x, dinv2, W1, b1, ga, Wg, bg, Wl1, bl1, Wl2, bl2):
    nsteps = N // BND
    full = lambda shape: pl.BlockSpec(shape, lambda i: tuple(0 for _ in shape))
    return pl.pallas_call(
        _tc_body,
        grid=(nsteps,),
        in_specs=[
            pl.BlockSpec((NSC, BND, FH), lambda i: (0, i, 0)),
            pl.BlockSpec((BND, F_IN), lambda i: (i, 0)),
            pl.BlockSpec((BND, 1), lambda i: (i, 0)),
            full((F_IN, H)),
            full((1, H)),
            full((1, 64)),
            full((64, H)),
            full((1, H)),
            full((2 * H, H)),
            full((1, H)),
            full((H, 2)),
            full((1, 2)),
        ],
        out_specs=pl.BlockSpec((1, 2), lambda i: (0, 0)),
        out_shape=jax.ShapeDtypeStruct((1, 2), jnp.float32),
        scratch_shapes=[pltpu.VMEM((1, H), jnp.float32)],
    )(s2, x, dinv2, W1, b1, ga, Wg, bg, Wl1, bl1, Wl2, bl2)


def kernel(x, edge_index, edge_attr, graph_attr, W1, b1, Wg, bg, Wl1, bl1,
           Wl2, bl2):
    if graph_attr.ndim == 1:
        graph_attr = graph_attr[None, :]
    src = edge_index[0]
    dst = edge_index[1]
    x2 = jnp.swapaxes(x.reshape(N, NSC, FH), 0, 1)
    s2, dinv = _sc_aggregate(x2, src, dst, edge_attr)
    return _tc_head(s2, x, dinv.reshape(N, 1), W1, b1.reshape(1, H),
                    graph_attr, Wg, bg.reshape(1, H), Wl1, bl1.reshape(1, H),
                    Wl2, bl2.reshape(1, 2))
